# speedup vs baseline: 3.0205x; 3.0205x over previous
import jax
import jax.numpy as jnp
from jax.experimental import pallas as pl


def _copy_block(in_ref, o_ref):
    o_ref[...] = in_ref[...]


def kernel(x, position_embeddings):
    seq_len = x.shape[1]
    d_model = position_embeddings.shape[1]
    block = 512
    out = pl.pallas_call(
        _copy_block,
        grid=(seq_len // block,),
        in_specs=[pl.BlockSpec((block, d_model), lambda i: (i, 0))],
        out_specs=pl.BlockSpec((block, d_model), lambda i: (i, 0)),
        out_shape=jax.ShapeDtypeStruct((seq_len, d_model), position_embeddings.dtype),
    )(position_embeddings)
    return out[None, :, :]
